# baseline (speedup 1.0000x reference)
import jax
import jax.numpy as jnp
from jax.experimental import pallas as pl
from jax.experimental.pallas import tpu as pltpu


def _oddeven_merge_sort_pairs(n):
    pairs = []

    def merge(lo, nn, r):
        step = r * 2
        if step < nn:
            merge(lo, nn, step)
            merge(lo + r, nn, step)
            for i in range(lo + r, lo + nn - r, step):
                pairs.append((i, i + r))
        else:
            pairs.append((lo, lo + r))

    def sort(lo, nn):
        if nn > 1:
            m = nn // 2
            sort(lo, m)
            sort(lo + m, m)
            merge(lo, nn, 1)

    sort(0, n)
    return pairs


_SORT16 = _oddeven_merge_sort_pairs(16)

_CHUNK = 512


def _vq_block_kernel(x_ref, v_ref, o_ref):
    v = v_ref[...]
    cols = [v[:, j : j + 1] for j in range(16)]
    for i, j in _SORT16:
        a, b = cols[i], cols[j]
        cols[i] = jnp.minimum(a, b)
        cols[j] = jnp.maximum(a, b)
    mids = [(cols[j] + cols[j + 1]) * 0.5 for j in range(15)]
    blk = x_ref.shape[1]

    def tree(x, lo, hi):
        if lo == hi:
            return jnp.broadcast_to(cols[lo], x.shape)
        mid = (lo + hi) // 2
        return jnp.where(x > mids[mid], tree(x, mid + 1, hi), tree(x, lo, mid))

    for c in range(0, blk, _CHUNK):
        x = x_ref[:, c : c + _CHUNK]
        o_ref[:, c : c + _CHUNK] = tree(x, 0, 15)


def kernel(x, values):
    R, L = x.shape
    BLK = 65536
    grid = (pl.cdiv(L, BLK),)
    out = pl.pallas_call(
        _vq_block_kernel,
        out_shape=jax.ShapeDtypeStruct((R, L), x.dtype),
        grid=grid,
        in_specs=[
            pl.BlockSpec((R, BLK), lambda i: (0, i)),
            pl.BlockSpec((R, 16), lambda i: (0, 0)),
        ],
        out_specs=pl.BlockSpec((R, BLK), lambda i: (0, i)),
        compiler_params=pltpu.CompilerParams(
            dimension_semantics=("parallel",),
        ),
    )(x, values)
    return out
